# Initial kernel scaffold; baseline (speedup 1.0000x reference)
import functools

import jax
import jax.numpy as jnp
from jax import lax
from jax.experimental import pallas as pl
from jax.experimental.pallas import tpu as pltpu
from jax.experimental.pallas import tpu_sc as plsc

N_NODES = 10000
N_EDGES = 320000
D_FEAT = 128
D_EDGE = 16
DH = 64
NC = 2
NS = 16
NW = NC * NS
EPW = N_EDGES // NW
CH = 80
NCHUNK = EPW // CH
RPW = N_NODES // NS
ZROWS = 125


def _tables_body(x_ref, we1_ref, we2_ref, xs0_ref, xs1_ref, xr0_ref, xr1_ref):
    xs = jnp.dot(x_ref[...], we1_ref[...], preferred_element_type=jnp.float32)
    xr = jnp.dot(x_ref[...], we2_ref[...], preferred_element_type=jnp.float32)
    xs0_ref[...] = xs[:, :DH]
    xs1_ref[...] = xs[:, DH:]
    xr0_ref[...] = xr[:, :DH]
    xr1_ref[...] = xr[:, DH:]


def _ea_body(ea_ref, we0_ref, be_ref, o0_ref, o1_ref):
    v = jnp.dot(ea_ref[...], we0_ref[...], preferred_element_type=jnp.float32)
    v = v + be_ref[...]
    o0_ref[...] = v[:, :DH]
    o1_ref[...] = v[:, DH:]


def _node_body(x_ref, accs_ref, accr_ref, cnts_ref, cntr_ref, wn_ref, bn_ref,
               o_ref):
    ss0 = accs_ref[0, 0] + accs_ref[0, 1]
    ss1 = accs_ref[1, 0] + accs_ref[1, 1]
    sr0 = accr_ref[0, 0] + accr_ref[0, 1]
    sr1 = accr_ref[1, 0] + accr_ref[1, 1]
    cs = jnp.sum(cnts_ref[...], axis=(0, 1))
    cr = jnp.sum(cntr_ref[...], axis=(0, 1))
    rs = (1.0 / jnp.maximum(cs, 1.0))[:, None]
    rr = (1.0 / jnp.maximum(cr, 1.0))[:, None]
    wn = wn_ref[...]
    out = jnp.dot(x_ref[...], wn[:128], preferred_element_type=jnp.float32)
    out += jnp.dot(ss0 * rs, wn[128:192], preferred_element_type=jnp.float32)
    out += jnp.dot(ss1 * rs, wn[192:256], preferred_element_type=jnp.float32)
    out += jnp.dot(sr0 * rr, wn[256:320], preferred_element_type=jnp.float32)
    out += jnp.dot(sr1 * rr, wn[320:384], preferred_element_type=jnp.float32)
    o_ref[...] = out + bn_ref[...]


def _sc_body(snd, rcv, ea0, ea1, xs0, xs1, xr0, xr1,
             ne, accs, accr, cnts, cntr,
             idx_s, idx_r, gs, gr, ob, cs_v, cr_v, zb,
             acc_s_sh, acc_r_sh, sem0, sem1):
    cid = lax.axis_index("c")
    sid = lax.axis_index("s")
    wid = sid * NC + cid
    ebase = wid * EPW
    rbase = sid * RPW

    zeros16 = jnp.zeros((16,), jnp.float32)
    ones16 = jnp.ones((16,), jnp.float32)

    @pl.loop(0, ZROWS)
    def _zz(r):
        for j in range(DH // 16):
            zb[r, pl.ds(j * 16, 16)] = zeros16

    @pl.loop(0, N_NODES // 16)
    def _zc(i):
        cs_v[pl.ds(i * 16, 16)] = zeros16
        cr_v[pl.ds(i * 16, 16)] = zeros16

    for h in (0, 1):
        ea_h = (ea0, ea1)[h]
        xs_h = (xs0, xs1)[h]
        xr_h = (xr0, xr1)[h]

        for j in range(RPW // ZROWS):
            r0 = rbase + j * ZROWS
            pltpu.sync_copy(zb, acc_s_sh.at[pl.ds(r0, ZROWS)])
            pltpu.sync_copy(zb, acc_r_sh.at[pl.ds(r0, ZROWS)])
        plsc.subcore_barrier()

        @pl.loop(0, NCHUNK)
        def _chunk(ci):
            base = ebase + ci * CH
            pltpu.sync_copy(snd.at[pl.ds(base, CH)], idx_s)
            pltpu.sync_copy(rcv.at[pl.ds(base, CH)], idx_r)
            g1 = pltpu.async_copy(xs_h.at[idx_s], gs, sem0)
            g2 = pltpu.async_copy(xr_h.at[idx_r], gr, sem1)
            pltpu.sync_copy(ea_h.at[pl.ds(base, CH)], ob)
            g1.wait()
            g2.wait()

            @pl.loop(0, CH)
            def _row(r):
                for j in range(DH // 16):
                    sl = pl.ds(j * 16, 16)
                    plsc.addupdate(ob.at[r, sl], gs[r, sl])
                    plsc.addupdate(ob.at[r, sl], gr[r, sl])

            pltpu.sync_copy(ob, ne.at[pl.ds(base, CH), pl.ds(h * DH, DH)])
            pltpu.sync_copy(ob, acc_s_sh.at[idx_s], add=True)
            pltpu.sync_copy(ob, acc_r_sh.at[idx_r], add=True)
            if h == 0:
                for k in range(CH // 16):
                    iv_s = idx_s[pl.ds(k * 16, 16)]
                    iv_r = idx_r[pl.ds(k * 16, 16)]
                    plsc.addupdate_scatter(cs_v, [iv_s], ones16)
                    plsc.addupdate_scatter(cr_v, [iv_r], ones16)

        plsc.subcore_barrier()
        for j in range(RPW // ZROWS):
            r0 = rbase + j * ZROWS
            pltpu.sync_copy(acc_s_sh.at[pl.ds(r0, ZROWS)],
                            accs.at[h, cid, pl.ds(r0, ZROWS)])
            pltpu.sync_copy(acc_r_sh.at[pl.ds(r0, ZROWS)],
                            accr.at[h, cid, pl.ds(r0, ZROWS)])

    pltpu.sync_copy(cs_v, cnts.at[cid, sid])
    pltpu.sync_copy(cr_v, cntr.at[cid, sid])


_sc_edges = functools.partial(
    pl.kernel,
    out_type=(
        jax.ShapeDtypeStruct((N_EDGES, D_FEAT), jnp.float32),
        jax.ShapeDtypeStruct((2, NC, N_NODES, DH), jnp.float32),
        jax.ShapeDtypeStruct((2, NC, N_NODES, DH), jnp.float32),
        jax.ShapeDtypeStruct((NC, NS, N_NODES), jnp.float32),
        jax.ShapeDtypeStruct((NC, NS, N_NODES), jnp.float32),
    ),
    mesh=plsc.VectorSubcoreMesh(core_axis_name="c", subcore_axis_name="s"),
    scratch_types=(
        pltpu.VMEM((CH,), jnp.int32),
        pltpu.VMEM((CH,), jnp.int32),
        pltpu.VMEM((CH, DH), jnp.float32),
        pltpu.VMEM((CH, DH), jnp.float32),
        pltpu.VMEM((CH, DH), jnp.float32),
        pltpu.VMEM((N_NODES,), jnp.float32),
        pltpu.VMEM((N_NODES,), jnp.float32),
        pltpu.VMEM((ZROWS, DH), jnp.float32),
        pltpu.VMEM_SHARED((N_NODES, DH), jnp.float32),
        pltpu.VMEM_SHARED((N_NODES, DH), jnp.float32),
        pltpu.SemaphoreType.DMA,
        pltpu.SemaphoreType.DMA,
    ),
)(_sc_body)


def kernel(x, edge_attr, senders, receivers, W_e, b_e, W_n, b_n):
    we0 = W_e[:D_EDGE]
    we1 = W_e[D_EDGE:D_EDGE + D_FEAT]
    we2 = W_e[D_EDGE + D_FEAT:]

    xs0, xs1, xr0, xr1 = pl.pallas_call(
        _tables_body,
        out_shape=[jax.ShapeDtypeStruct((N_NODES, DH), jnp.float32)] * 4,
    )(x, we1, we2)

    BE = 8000
    ea0, ea1 = pl.pallas_call(
        _ea_body,
        grid=(N_EDGES // BE,),
        in_specs=[
            pl.BlockSpec((BE, D_EDGE), lambda i: (i, 0)),
            pl.BlockSpec((D_EDGE, D_FEAT), lambda i: (0, 0)),
            pl.BlockSpec((1, D_FEAT), lambda i: (0, 0)),
        ],
        out_specs=[
            pl.BlockSpec((BE, DH), lambda i: (i, 0)),
            pl.BlockSpec((BE, DH), lambda i: (i, 0)),
        ],
        out_shape=[jax.ShapeDtypeStruct((N_EDGES, DH), jnp.float32)] * 2,
    )(edge_attr, we0, b_e.reshape(1, D_FEAT))

    ne, accs, accr, cnts, cntr = _sc_edges(
        senders, receivers, ea0, ea1, xs0, xs1, xr0, xr1)

    BN = 400
    new_nodes = pl.pallas_call(
        _node_body,
        grid=(N_NODES // BN,),
        in_specs=[
            pl.BlockSpec((BN, D_FEAT), lambda i: (i, 0)),
            pl.BlockSpec((2, NC, BN, DH), lambda i: (0, 0, i, 0)),
            pl.BlockSpec((2, NC, BN, DH), lambda i: (0, 0, i, 0)),
            pl.BlockSpec((NC, NS, BN), lambda i: (0, 0, i)),
            pl.BlockSpec((NC, NS, BN), lambda i: (0, 0, i)),
            pl.BlockSpec((3 * D_FEAT, D_FEAT), lambda i: (0, 0)),
            pl.BlockSpec((1, D_FEAT), lambda i: (0, 0)),
        ],
        out_specs=pl.BlockSpec((BN, D_FEAT), lambda i: (i, 0)),
        out_shape=jax.ShapeDtypeStruct((N_NODES, D_FEAT), jnp.float32),
    )(x, accs, accr, cnts, cntr, W_n, b_n.reshape(1, D_FEAT))

    return new_nodes, ne


# speedup vs baseline: 2.0886x; 2.0886x over previous
import functools

import jax
import jax.numpy as jnp
from jax import lax
from jax.experimental import pallas as pl
from jax.experimental.pallas import tpu as pltpu
from jax.experimental.pallas import tpu_sc as plsc

N_NODES = 10000
N_EDGES = 320000
D_FEAT = 128
D_EDGE = 16
DH = 64
NC = 2
NS = 16
NW = NC * NS
EPW = N_EDGES // NW
CH = 80
NCHUNK = EPW // CH
N_PAD = 10000
RPW = N_PAD // NS
ZROWS = 125


def _tables_body(x_ref, we1_ref, we2_ref, xs0_ref, xs1_ref, xr0_ref, xr1_ref):
    xs = jnp.dot(x_ref[...], we1_ref[...], preferred_element_type=jnp.float32)
    xr = jnp.dot(x_ref[...], we2_ref[...], preferred_element_type=jnp.float32)
    xs0_ref[...] = xs[:, :DH]
    xs1_ref[...] = xs[:, DH:]
    xr0_ref[...] = xr[:, :DH]
    xr1_ref[...] = xr[:, DH:]


def _ea_body(ea_ref, we0_ref, be_ref, o0_ref, o1_ref):
    v = jnp.dot(ea_ref[...], we0_ref[...], preferred_element_type=jnp.float32)
    v = v + be_ref[...]
    o0_ref[...] = v[:, :DH]
    o1_ref[...] = v[:, DH:]


def _ilv_body(ne_ref, o_ref):
    o_ref[:, :DH] = ne_ref[0]
    o_ref[:, DH:] = ne_ref[1]


def _node_body(x_ref, accs_ref, accr_ref, cnts_ref, cntr_ref, wn_ref, bn_ref,
               o_ref):
    ss0 = accs_ref[0, 0] + accs_ref[0, 1]
    ss1 = accs_ref[1, 0] + accs_ref[1, 1]
    sr0 = accr_ref[0, 0] + accr_ref[0, 1]
    sr1 = accr_ref[1, 0] + accr_ref[1, 1]
    cs = cnts_ref[0, :, :1] + cnts_ref[1, :, :1]
    cr = cntr_ref[0, :, :1] + cntr_ref[1, :, :1]
    rs = 1.0 / jnp.maximum(cs, 1.0)
    rr = 1.0 / jnp.maximum(cr, 1.0)
    wn = wn_ref[...]
    out = jnp.dot(x_ref[...], wn[:128], preferred_element_type=jnp.float32)
    out += jnp.dot(ss0 * rs, wn[128:192], preferred_element_type=jnp.float32)
    out += jnp.dot(ss1 * rs, wn[192:256], preferred_element_type=jnp.float32)
    out += jnp.dot(sr0 * rr, wn[256:320], preferred_element_type=jnp.float32)
    out += jnp.dot(sr1 * rr, wn[320:384], preferred_element_type=jnp.float32)
    o_ref[...] = out + bn_ref[...]


def _sc_body(snd, rcv, ea0, ea1, xs0, xs1, xr0, xr1,
             ne, accs, accr, cnts, cntr,
             idx_s, idx_r, gs, gr, ob, ones_b, zbc, zb,
             acc_s_sh, acc_r_sh, cnt_s_sh, cnt_r_sh, sem0, sem1):
    cid = lax.axis_index("c")
    sid = lax.axis_index("s")
    wid = sid * NC + cid
    ebase = wid * EPW
    rbase = sid * RPW

    zeros16 = jnp.zeros((16,), jnp.float32)
    ones16 = jnp.ones((16,), jnp.float32)

    @pl.loop(0, ZROWS)
    def _zz(r):
        for j in range(DH // 16):
            zb[r, pl.ds(j * 16, 16)] = zeros16
        zbc[r, pl.ds(0, 16)] = zeros16

    @pl.loop(0, CH)
    def _zo(r):
        ones_b[r, pl.ds(0, 16)] = ones16

    for j in range(RPW // ZROWS):
        r0 = rbase + j * ZROWS
        pltpu.sync_copy(zbc, cnt_s_sh.at[pl.ds(r0, ZROWS)])
        pltpu.sync_copy(zbc, cnt_r_sh.at[pl.ds(r0, ZROWS)])

    for h in (0, 1):
        ea_h = (ea0, ea1)[h]
        xs_h = (xs0, xs1)[h]
        xr_h = (xr0, xr1)[h]

        for j in range(RPW // ZROWS):
            r0 = rbase + j * ZROWS
            pltpu.sync_copy(zb, acc_s_sh.at[pl.ds(r0, ZROWS)])
            pltpu.sync_copy(zb, acc_r_sh.at[pl.ds(r0, ZROWS)])
        plsc.subcore_barrier()

        @pl.loop(0, NCHUNK)
        def _chunk(ci):
            base = ebase + ci * CH
            pltpu.sync_copy(snd.at[pl.ds(base, CH)], idx_s)
            pltpu.sync_copy(rcv.at[pl.ds(base, CH)], idx_r)
            g1 = pltpu.async_copy(xs_h.at[idx_s], gs, sem0)
            g2 = pltpu.async_copy(xr_h.at[idx_r], gr, sem1)
            pltpu.sync_copy(ea_h.at[pl.ds(base, CH)], ob)
            g1.wait()
            g2.wait()

            @pl.loop(0, CH)
            def _row(r):
                for j in range(DH // 16):
                    sl = pl.ds(j * 16, 16)
                    plsc.addupdate(ob.at[r, sl], gs[r, sl])
                    plsc.addupdate(ob.at[r, sl], gr[r, sl])

            pltpu.sync_copy(ob, ne.at[h, pl.ds(base, CH)])
            pltpu.sync_copy(ob, acc_s_sh.at[idx_s], add=True)
            pltpu.sync_copy(ob, acc_r_sh.at[idx_r], add=True)
            if h == 0:
                pltpu.sync_copy(ones_b, cnt_s_sh.at[idx_s], add=True)
                pltpu.sync_copy(ones_b, cnt_r_sh.at[idx_r], add=True)

        plsc.subcore_barrier()
        for j in range(RPW // ZROWS):
            r0 = rbase + j * ZROWS
            pltpu.sync_copy(acc_s_sh.at[pl.ds(r0, ZROWS)],
                            accs.at[h, cid, pl.ds(r0, ZROWS)])
            pltpu.sync_copy(acc_r_sh.at[pl.ds(r0, ZROWS)],
                            accr.at[h, cid, pl.ds(r0, ZROWS)])
        if h == 0:
            for j in range(RPW // ZROWS):
                r0 = rbase + j * ZROWS
                pltpu.sync_copy(cnt_s_sh.at[pl.ds(r0, ZROWS)],
                                cnts.at[cid, pl.ds(r0, ZROWS)])
                pltpu.sync_copy(cnt_r_sh.at[pl.ds(r0, ZROWS)],
                                cntr.at[cid, pl.ds(r0, ZROWS)])


_sc_edges = functools.partial(
    pl.kernel,
    out_type=(
        jax.ShapeDtypeStruct((2, N_EDGES, DH), jnp.float32),
        jax.ShapeDtypeStruct((2, NC, N_PAD, DH), jnp.float32),
        jax.ShapeDtypeStruct((2, NC, N_PAD, DH), jnp.float32),
        jax.ShapeDtypeStruct((NC, N_PAD, 16), jnp.float32),
        jax.ShapeDtypeStruct((NC, N_PAD, 16), jnp.float32),
    ),
    mesh=plsc.VectorSubcoreMesh(core_axis_name="c", subcore_axis_name="s"),
    compiler_params=pltpu.CompilerParams(use_tc_tiling_on_sc=False),
    scratch_types=(
        pltpu.VMEM((CH,), jnp.int32),
        pltpu.VMEM((CH,), jnp.int32),
        pltpu.VMEM((CH, DH), jnp.float32),
        pltpu.VMEM((CH, DH), jnp.float32),
        pltpu.VMEM((CH, DH), jnp.float32),
        pltpu.VMEM((CH, 16), jnp.float32),
        pltpu.VMEM((ZROWS, 16), jnp.float32),
        pltpu.VMEM((ZROWS, DH), jnp.float32),
        pltpu.VMEM_SHARED((N_PAD, DH), jnp.float32),
        pltpu.VMEM_SHARED((N_PAD, DH), jnp.float32),
        pltpu.VMEM_SHARED((N_PAD, 16), jnp.float32),
        pltpu.VMEM_SHARED((N_PAD, 16), jnp.float32),
        pltpu.SemaphoreType.DMA,
        pltpu.SemaphoreType.DMA,
    ),
)(_sc_body)


def kernel(x, edge_attr, senders, receivers, W_e, b_e, W_n, b_n):
    we0 = W_e[:D_EDGE]
    we1 = W_e[D_EDGE:D_EDGE + D_FEAT]
    we2 = W_e[D_EDGE + D_FEAT:]

    xs0, xs1, xr0, xr1 = pl.pallas_call(
        _tables_body,
        out_shape=[jax.ShapeDtypeStruct((N_NODES, DH), jnp.float32)] * 4,
    )(x, we1, we2)

    BE = 8000
    ea0, ea1 = pl.pallas_call(
        _ea_body,
        grid=(N_EDGES // BE,),
        in_specs=[
            pl.BlockSpec((BE, D_EDGE), lambda i: (i, 0)),
            pl.BlockSpec((D_EDGE, D_FEAT), lambda i: (0, 0)),
            pl.BlockSpec((1, D_FEAT), lambda i: (0, 0)),
        ],
        out_specs=[
            pl.BlockSpec((BE, DH), lambda i: (i, 0)),
            pl.BlockSpec((BE, DH), lambda i: (i, 0)),
        ],
        out_shape=[jax.ShapeDtypeStruct((N_EDGES, DH), jnp.float32)] * 2,
    )(edge_attr, we0, b_e.reshape(1, D_FEAT))

    ne, accs, accr, cnts, cntr = _sc_edges(
        senders, receivers, ea0, ea1, xs0, xs1, xr0, xr1)

    BI = 16000
    new_edges = pl.pallas_call(
        _ilv_body,
        grid=(N_EDGES // BI,),
        in_specs=[pl.BlockSpec((2, BI, DH), lambda i: (0, i, 0))],
        out_specs=pl.BlockSpec((BI, D_FEAT), lambda i: (i, 0)),
        out_shape=jax.ShapeDtypeStruct((N_EDGES, D_FEAT), jnp.float32),
    )(ne)

    BN = 400
    new_nodes = pl.pallas_call(
        _node_body,
        grid=(N_NODES // BN,),
        in_specs=[
            pl.BlockSpec((BN, D_FEAT), lambda i: (i, 0)),
            pl.BlockSpec((2, NC, BN, DH), lambda i: (0, 0, i, 0)),
            pl.BlockSpec((2, NC, BN, DH), lambda i: (0, 0, i, 0)),
            pl.BlockSpec((NC, BN, 16), lambda i: (0, i, 0)),
            pl.BlockSpec((NC, BN, 16), lambda i: (0, i, 0)),
            pl.BlockSpec((3 * D_FEAT, D_FEAT), lambda i: (0, 0)),
            pl.BlockSpec((1, D_FEAT), lambda i: (0, 0)),
        ],
        out_specs=pl.BlockSpec((BN, D_FEAT), lambda i: (i, 0)),
        out_shape=jax.ShapeDtypeStruct((N_NODES, D_FEAT), jnp.float32),
    )(x, accs, accr, cnts, cntr, W_n, b_n.reshape(1, D_FEAT))

    return new_nodes, new_edges


# speedup vs baseline: 3.2927x; 1.5765x over previous
import functools

import jax
import jax.numpy as jnp
from jax import lax
from jax.experimental import pallas as pl
from jax.experimental.pallas import tpu as pltpu
from jax.experimental.pallas import tpu_sc as plsc

N_NODES = 10000
N_EDGES = 320000
D_FEAT = 128
D_EDGE = 16
NC = 2
NS = 16
NW = NC * NS
EPW = N_EDGES // NW
CH = 80
NCHUNK = EPW // CH
RPW = N_NODES // NS
ZROWS = 25


def _tables_body(x_ref, we1_ref, we2_ref, xs_ref, xr_ref):
    xs_ref[...] = jnp.dot(x_ref[...], we1_ref[...],
                          preferred_element_type=jnp.float32)
    xr_ref[...] = jnp.dot(x_ref[...], we2_ref[...],
                          preferred_element_type=jnp.float32)


def _ea_body(ea_ref, we0_ref, be_ref, o_ref):
    o_ref[...] = jnp.dot(ea_ref[...], we0_ref[...],
                         preferred_element_type=jnp.float32) + be_ref[...]


def _node_body(x_ref, accs_ref, accr_ref, cnts_ref, cntr_ref, wn_ref, bn_ref,
               o_ref):
    ss = accs_ref[0] + accs_ref[1]
    sr = accr_ref[0] + accr_ref[1]
    cs = cnts_ref[0, :, :1] + cnts_ref[1, :, :1]
    cr = cntr_ref[0, :, :1] + cntr_ref[1, :, :1]
    rs = 1.0 / jnp.maximum(cs, 1.0)
    rr = 1.0 / jnp.maximum(cr, 1.0)
    wn = wn_ref[...]
    out = jnp.dot(x_ref[...], wn[:128], preferred_element_type=jnp.float32)
    out += jnp.dot(ss * rs, wn[128:256], preferred_element_type=jnp.float32)
    out += jnp.dot(sr * rr, wn[256:384], preferred_element_type=jnp.float32)
    o_ref[...] = out + bn_ref[...]


def _sc_body(snd, rcv, ea, xst, xrt,
             ne, accs, accr, cnts, cntr,
             idx_s, idx_r, gs, gr, ob, ones_b, zbc, zb,
             acc_sh, cnt_sh, sem0, sem1):
    cid = lax.axis_index("c")
    sid = lax.axis_index("s")
    wid = sid * NC + cid
    ebase = wid * EPW
    rbase = sid * RPW

    zeros16 = jnp.zeros((16,), jnp.float32)
    ones16 = jnp.ones((16,), jnp.float32)

    @pl.loop(0, ZROWS)
    def _zz(r):
        for j in range(D_FEAT // 16):
            zb[r, pl.ds(j * 16, 16)] = zeros16
        zbc[r, pl.ds(0, 16)] = zeros16

    @pl.loop(0, CH)
    def _zo(r):
        ones_b[r, pl.ds(0, 16)] = ones16

    for j in range(RPW // ZROWS):
        r0 = rbase + j * ZROWS
        pltpu.sync_copy(zbc, cnt_sh.at[pl.ds(r0, ZROWS)])
        pltpu.sync_copy(zb, acc_sh.at[pl.ds(r0, ZROWS)])
    plsc.subcore_barrier()

    @pl.loop(0, NCHUNK)
    def _chunk(ci):
        base = ebase + ci * CH
        pltpu.sync_copy(snd.at[pl.ds(base, CH)], idx_s)
        pltpu.sync_copy(rcv.at[pl.ds(base, CH)], idx_r)
        g1 = pltpu.async_copy(xst.at[idx_s], gs, sem0)
        g2 = pltpu.async_copy(xrt.at[idx_r], gr, sem1)
        pltpu.sync_copy(ea.at[pl.ds(base, CH)], ob)
        g1.wait()
        g2.wait()

        @pl.loop(0, CH)
        def _row(r):
            for j in range(D_FEAT // 16):
                sl = pl.ds(j * 16, 16)
                plsc.addupdate(ob.at[r, sl], gs[r, sl])
                plsc.addupdate(ob.at[r, sl], gr[r, sl])

        pltpu.sync_copy(ob, ne.at[pl.ds(base, CH)])
        pltpu.sync_copy(ob, acc_sh.at[idx_s], add=True)
        pltpu.sync_copy(ones_b, cnt_sh.at[idx_s], add=True)

    plsc.subcore_barrier()
    for j in range(RPW // ZROWS):
        r0 = rbase + j * ZROWS
        pltpu.sync_copy(acc_sh.at[pl.ds(r0, ZROWS)],
                        accs.at[cid, pl.ds(r0, ZROWS)])
        pltpu.sync_copy(cnt_sh.at[pl.ds(r0, ZROWS)],
                        cnts.at[cid, pl.ds(r0, ZROWS)])
        pltpu.sync_copy(zb, acc_sh.at[pl.ds(r0, ZROWS)])
        pltpu.sync_copy(zbc, cnt_sh.at[pl.ds(r0, ZROWS)])
    plsc.subcore_barrier()

    @pl.loop(0, NCHUNK)
    def _chunk_b(ci):
        base = ebase + ci * CH
        pltpu.sync_copy(rcv.at[pl.ds(base, CH)], idx_r)
        pltpu.sync_copy(ne.at[pl.ds(base, CH)], ob)
        pltpu.sync_copy(ob, acc_sh.at[idx_r], add=True)
        pltpu.sync_copy(ones_b, cnt_sh.at[idx_r], add=True)

    plsc.subcore_barrier()
    for j in range(RPW // ZROWS):
        r0 = rbase + j * ZROWS
        pltpu.sync_copy(acc_sh.at[pl.ds(r0, ZROWS)],
                        accr.at[cid, pl.ds(r0, ZROWS)])
        pltpu.sync_copy(cnt_sh.at[pl.ds(r0, ZROWS)],
                        cntr.at[cid, pl.ds(r0, ZROWS)])


_sc_edges = functools.partial(
    pl.kernel,
    out_type=(
        jax.ShapeDtypeStruct((N_EDGES, D_FEAT), jnp.float32),
        jax.ShapeDtypeStruct((NC, N_NODES, D_FEAT), jnp.float32),
        jax.ShapeDtypeStruct((NC, N_NODES, D_FEAT), jnp.float32),
        jax.ShapeDtypeStruct((NC, N_NODES, 16), jnp.float32),
        jax.ShapeDtypeStruct((NC, N_NODES, 16), jnp.float32),
    ),
    mesh=plsc.VectorSubcoreMesh(core_axis_name="c", subcore_axis_name="s"),
    compiler_params=pltpu.CompilerParams(use_tc_tiling_on_sc=False),
    scratch_types=(
        pltpu.VMEM((CH,), jnp.int32),
        pltpu.VMEM((CH,), jnp.int32),
        pltpu.VMEM((CH, D_FEAT), jnp.float32),
        pltpu.VMEM((CH, D_FEAT), jnp.float32),
        pltpu.VMEM((CH, D_FEAT), jnp.float32),
        pltpu.VMEM((CH, 16), jnp.float32),
        pltpu.VMEM((ZROWS, 16), jnp.float32),
        pltpu.VMEM((ZROWS, D_FEAT), jnp.float32),
        pltpu.VMEM_SHARED((N_NODES, D_FEAT), jnp.float32),
        pltpu.VMEM_SHARED((N_NODES, 16), jnp.float32),
        pltpu.SemaphoreType.DMA,
        pltpu.SemaphoreType.DMA,
    ),
)(_sc_body)


def kernel(x, edge_attr, senders, receivers, W_e, b_e, W_n, b_n):
    we0 = W_e[:D_EDGE]
    we1 = W_e[D_EDGE:D_EDGE + D_FEAT]
    we2 = W_e[D_EDGE + D_FEAT:]

    xst, xrt = pl.pallas_call(
        _tables_body,
        out_shape=[jax.ShapeDtypeStruct((N_NODES, D_FEAT), jnp.float32)] * 2,
    )(x, we1, we2)

    BE = 8000
    ea = pl.pallas_call(
        _ea_body,
        grid=(N_EDGES // BE,),
        in_specs=[
            pl.BlockSpec((BE, D_EDGE), lambda i: (i, 0)),
            pl.BlockSpec((D_EDGE, D_FEAT), lambda i: (0, 0)),
            pl.BlockSpec((1, D_FEAT), lambda i: (0, 0)),
        ],
        out_specs=pl.BlockSpec((BE, D_FEAT), lambda i: (i, 0)),
        out_shape=jax.ShapeDtypeStruct((N_EDGES, D_FEAT), jnp.float32),
    )(edge_attr, we0, b_e.reshape(1, D_FEAT))

    ne, accs, accr, cnts, cntr = _sc_edges(senders, receivers, ea, xst, xrt)

    BN = 400
    new_nodes = pl.pallas_call(
        _node_body,
        grid=(N_NODES // BN,),
        in_specs=[
            pl.BlockSpec((BN, D_FEAT), lambda i: (i, 0)),
            pl.BlockSpec((NC, BN, D_FEAT), lambda i: (0, i, 0)),
            pl.BlockSpec((NC, BN, D_FEAT), lambda i: (0, i, 0)),
            pl.BlockSpec((NC, BN, 16), lambda i: (0, i, 0)),
            pl.BlockSpec((NC, BN, 16), lambda i: (0, i, 0)),
            pl.BlockSpec((3 * D_FEAT, D_FEAT), lambda i: (0, 0)),
            pl.BlockSpec((1, D_FEAT), lambda i: (0, 0)),
        ],
        out_specs=pl.BlockSpec((BN, D_FEAT), lambda i: (i, 0)),
        out_shape=jax.ShapeDtypeStruct((N_NODES, D_FEAT), jnp.float32),
    )(x, accs, accr, cnts, cntr, W_n, b_n.reshape(1, D_FEAT))

    return new_nodes, ne


# speedup vs baseline: 3.8648x; 1.1737x over previous
import functools

import jax
import jax.numpy as jnp
from jax import lax
from jax.experimental import pallas as pl
from jax.experimental.pallas import tpu as pltpu
from jax.experimental.pallas import tpu_sc as plsc

N_NODES = 10000
N_EDGES = 320000
D_FEAT = 128
D_EDGE = 16
NC = 2
NS = 16
NW = NC * NS
EPW = N_EDGES // NW
CH = 80
NCHUNK = EPW // CH
RPW = N_NODES // NS
ZROWS = 25


def _tables_body(x_ref, we1_ref, we2_ref, xs_ref, xr_ref):
    xs_ref[...] = jnp.dot(x_ref[...], we1_ref[...],
                          preferred_element_type=jnp.float32)
    xr_ref[...] = jnp.dot(x_ref[...], we2_ref[...],
                          preferred_element_type=jnp.float32)


def _ea_body(ea_ref, we0_ref, be_ref, o_ref):
    o_ref[...] = jnp.dot(ea_ref[...], we0_ref[...],
                         preferred_element_type=jnp.float32) + be_ref[...]


def _node_body(x_ref, accs_ref, accr_ref, cnts_ref, cntr_ref, wn_ref, bn_ref,
               o_ref):
    ss = accs_ref[0] + accs_ref[1]
    sr = accr_ref[0] + accr_ref[1]
    cs = cnts_ref[0, :, :1] + cnts_ref[1, :, :1]
    cr = cntr_ref[0, :, :1] + cntr_ref[1, :, :1]
    rs = 1.0 / jnp.maximum(cs, 1.0)
    rr = 1.0 / jnp.maximum(cr, 1.0)
    wn = wn_ref[...]
    out = jnp.dot(x_ref[...], wn[:128], preferred_element_type=jnp.float32)
    out += jnp.dot(ss * rs, wn[128:256], preferred_element_type=jnp.float32)
    out += jnp.dot(sr * rr, wn[256:384], preferred_element_type=jnp.float32)
    o_ref[...] = out + bn_ref[...]


def _sc_body(snd, rcv, ea, xst, xrt, zacc, zcnt,
             ne, accs, accr, cnts, cntr,
             idx_s, idx_r, gs, gr, ob, ones_b,
             acc_sh, cnt_sh, semi, semg, semw, semz):
    cid = lax.axis_index("c")
    sid = lax.axis_index("s")
    wid = sid * NC + cid
    ebase = wid * EPW
    rbase = sid * RPW

    ones16 = jnp.ones((16,), jnp.float32)

    @pl.loop(0, CH)
    def _zo(r):
        ones_b[r, pl.ds(0, 16)] = ones16

    z1 = pltpu.async_copy(zacc, acc_sh.at[pl.ds(rbase, RPW)], semz)
    z2 = pltpu.async_copy(zcnt, cnt_sh.at[pl.ds(rbase, RPW)], semz)
    z1.wait()
    z2.wait()
    plsc.subcore_barrier()

    @pl.loop(0, NCHUNK)
    def _chunk(ci):
        base = ebase + ci * CH
        i1 = pltpu.async_copy(snd.at[pl.ds(base, CH)], idx_s, semi)
        i2 = pltpu.async_copy(rcv.at[pl.ds(base, CH)], idx_r, semi)
        i1.wait()
        i2.wait()
        g1 = pltpu.async_copy(xst.at[idx_s], gs, semg)
        g2 = pltpu.async_copy(xrt.at[idx_r], gr, semg)
        g3 = pltpu.async_copy(ea.at[pl.ds(base, CH)], ob, semg)
        g1.wait()
        g2.wait()
        g3.wait()

        @pl.loop(0, CH)
        def _row(r):
            for j in range(D_FEAT // 16):
                sl = pl.ds(j * 16, 16)
                plsc.addupdate(ob.at[r, sl], gs[r, sl])
                plsc.addupdate(ob.at[r, sl], gr[r, sl])

        w1 = pltpu.async_copy(ob, ne.at[pl.ds(base, CH)], semw)
        pltpu.sync_copy(ob, acc_sh.at[idx_s], add=True)
        pltpu.sync_copy(ones_b, cnt_sh.at[idx_s], add=True)
        w1.wait()

    plsc.subcore_barrier()
    f1 = pltpu.async_copy(acc_sh.at[pl.ds(rbase, RPW)],
                          accs.at[cid, pl.ds(rbase, RPW)], semz)
    f2 = pltpu.async_copy(cnt_sh.at[pl.ds(rbase, RPW)],
                          cnts.at[cid, pl.ds(rbase, RPW)], semz)
    f1.wait()
    f2.wait()
    z1 = pltpu.async_copy(zacc, acc_sh.at[pl.ds(rbase, RPW)], semz)
    z2 = pltpu.async_copy(zcnt, cnt_sh.at[pl.ds(rbase, RPW)], semz)
    z1.wait()
    z2.wait()
    plsc.subcore_barrier()

    @pl.loop(0, NCHUNK)
    def _chunk_b(ci):
        base = ebase + ci * CH
        i1 = pltpu.async_copy(rcv.at[pl.ds(base, CH)], idx_r, semi)
        i2 = pltpu.async_copy(ne.at[pl.ds(base, CH)], ob, semg)
        i1.wait()
        i2.wait()
        pltpu.sync_copy(ob, acc_sh.at[idx_r], add=True)
        pltpu.sync_copy(ones_b, cnt_sh.at[idx_r], add=True)

    plsc.subcore_barrier()
    f1 = pltpu.async_copy(acc_sh.at[pl.ds(rbase, RPW)],
                          accr.at[cid, pl.ds(rbase, RPW)], semz)
    f2 = pltpu.async_copy(cnt_sh.at[pl.ds(rbase, RPW)],
                          cntr.at[cid, pl.ds(rbase, RPW)], semz)
    f1.wait()
    f2.wait()


_sc_edges = functools.partial(
    pl.kernel,
    out_type=(
        jax.ShapeDtypeStruct((N_EDGES, D_FEAT), jnp.float32),
        jax.ShapeDtypeStruct((NC, N_NODES, D_FEAT), jnp.float32),
        jax.ShapeDtypeStruct((NC, N_NODES, D_FEAT), jnp.float32),
        jax.ShapeDtypeStruct((NC, N_NODES, 16), jnp.float32),
        jax.ShapeDtypeStruct((NC, N_NODES, 16), jnp.float32),
    ),
    mesh=plsc.VectorSubcoreMesh(core_axis_name="c", subcore_axis_name="s"),
    compiler_params=pltpu.CompilerParams(use_tc_tiling_on_sc=False),
    scratch_types=(
        pltpu.VMEM((CH,), jnp.int32),
        pltpu.VMEM((CH,), jnp.int32),
        pltpu.VMEM((CH, D_FEAT), jnp.float32),
        pltpu.VMEM((CH, D_FEAT), jnp.float32),
        pltpu.VMEM((CH, D_FEAT), jnp.float32),
        pltpu.VMEM((CH, 16), jnp.float32),
        pltpu.VMEM_SHARED((N_NODES, D_FEAT), jnp.float32),
        pltpu.VMEM_SHARED((N_NODES, 16), jnp.float32),
        pltpu.SemaphoreType.DMA,
        pltpu.SemaphoreType.DMA,
        pltpu.SemaphoreType.DMA,
        pltpu.SemaphoreType.DMA,
    ),
)(_sc_body)


def kernel(x, edge_attr, senders, receivers, W_e, b_e, W_n, b_n):
    we0 = W_e[:D_EDGE]
    we1 = W_e[D_EDGE:D_EDGE + D_FEAT]
    we2 = W_e[D_EDGE + D_FEAT:]

    xst, xrt = pl.pallas_call(
        _tables_body,
        out_shape=[jax.ShapeDtypeStruct((N_NODES, D_FEAT), jnp.float32)] * 2,
    )(x, we1, we2)

    BE = 8000
    ea = pl.pallas_call(
        _ea_body,
        grid=(N_EDGES // BE,),
        in_specs=[
            pl.BlockSpec((BE, D_EDGE), lambda i: (i, 0)),
            pl.BlockSpec((D_EDGE, D_FEAT), lambda i: (0, 0)),
            pl.BlockSpec((1, D_FEAT), lambda i: (0, 0)),
        ],
        out_specs=pl.BlockSpec((BE, D_FEAT), lambda i: (i, 0)),
        out_shape=jax.ShapeDtypeStruct((N_EDGES, D_FEAT), jnp.float32),
    )(edge_attr, we0, b_e.reshape(1, D_FEAT))

    zacc = jnp.zeros((RPW, D_FEAT), jnp.float32)
    zcnt = jnp.zeros((RPW, 16), jnp.float32)
    ne, accs, accr, cnts, cntr = _sc_edges(
        senders, receivers, ea, xst, xrt, zacc, zcnt)

    BN = 400
    new_nodes = pl.pallas_call(
        _node_body,
        grid=(N_NODES // BN,),
        in_specs=[
            pl.BlockSpec((BN, D_FEAT), lambda i: (i, 0)),
            pl.BlockSpec((NC, BN, D_FEAT), lambda i: (0, i, 0)),
            pl.BlockSpec((NC, BN, D_FEAT), lambda i: (0, i, 0)),
            pl.BlockSpec((NC, BN, 16), lambda i: (0, i, 0)),
            pl.BlockSpec((NC, BN, 16), lambda i: (0, i, 0)),
            pl.BlockSpec((3 * D_FEAT, D_FEAT), lambda i: (0, 0)),
            pl.BlockSpec((1, D_FEAT), lambda i: (0, 0)),
        ],
        out_specs=pl.BlockSpec((BN, D_FEAT), lambda i: (i, 0)),
        out_shape=jax.ShapeDtypeStruct((N_NODES, D_FEAT), jnp.float32),
    )(x, accs, accr, cnts, cntr, W_n, b_n.reshape(1, D_FEAT))

    return new_nodes, ne


# speedup vs baseline: 3.9220x; 1.0148x over previous
import functools

import jax
import jax.numpy as jnp
from jax import lax
from jax.experimental import pallas as pl
from jax.experimental.pallas import tpu as pltpu
from jax.experimental.pallas import tpu_sc as plsc

N_NODES = 10000
N_EDGES = 320000
D_FEAT = 128
D_EDGE = 16
NC = 2
NS = 16
NW = NC * NS
EPW = N_EDGES // NW
CH = 80
NCHUNK = EPW // CH
RPW = N_NODES // NS
ZROWS = 25


def _tables_body(x_ref, we1_ref, we2_ref, xs_ref, xr_ref):
    xs_ref[...] = jnp.dot(x_ref[...], we1_ref[...],
                          preferred_element_type=jnp.float32)
    xr_ref[...] = jnp.dot(x_ref[...], we2_ref[...],
                          preferred_element_type=jnp.float32)


def _ea_body(ea_ref, we0_ref, be_ref, o_ref):
    o_ref[...] = jnp.dot(ea_ref[...], we0_ref[...],
                         preferred_element_type=jnp.float32) + be_ref[...]


def _node_body(x_ref, accs_ref, accr_ref, cnts_ref, cntr_ref, wn_ref, bn_ref,
               o_ref):
    ss = accs_ref[0] + accs_ref[1]
    sr = accr_ref[0] + accr_ref[1]
    cs = cnts_ref[0, :, :1] + cnts_ref[1, :, :1]
    cr = cntr_ref[0, :, :1] + cntr_ref[1, :, :1]
    rs = 1.0 / jnp.maximum(cs, 1.0)
    rr = 1.0 / jnp.maximum(cr, 1.0)
    wn = wn_ref[...]
    out = jnp.dot(x_ref[...], wn[:128], preferred_element_type=jnp.float32)
    out += jnp.dot(ss * rs, wn[128:256], preferred_element_type=jnp.float32)
    out += jnp.dot(sr * rr, wn[256:384], preferred_element_type=jnp.float32)
    o_ref[...] = out + bn_ref[...]


def _sc_body(snd, rcv, ea, xst, xrt, zacc, zcnt,
             ne, accs, accr, cnts, cntr,
             idx_s, idx_r, gs, gr, ob, ones_b,
             acc_sh, cnt_sh, semi, semg, semw, semz, sema, semc):
    cid = lax.axis_index("c")
    sid = lax.axis_index("s")
    wid = sid * NC + cid
    ebase = wid * EPW
    rbase = sid * RPW

    ones16 = jnp.ones((16,), jnp.float32)

    @pl.loop(0, CH)
    def _zo(r):
        ones_b[r, pl.ds(0, 16)] = ones16

    z1 = pltpu.async_copy(zacc, acc_sh.at[pl.ds(rbase, RPW)], semz)
    z2 = pltpu.async_copy(zcnt, cnt_sh.at[pl.ds(rbase, RPW)], semz)
    z1.wait()
    z2.wait()
    plsc.subcore_barrier()

    @pl.loop(0, NCHUNK)
    def _chunk(ci):
        base = ebase + ci * CH
        i1 = pltpu.async_copy(snd.at[pl.ds(base, CH)], idx_s, semi)
        i2 = pltpu.async_copy(rcv.at[pl.ds(base, CH)], idx_r, semi)
        i1.wait()
        i2.wait()
        g1 = pltpu.async_copy(xst.at[idx_s], gs, semg)
        g2 = pltpu.async_copy(xrt.at[idx_r], gr, semg)
        g3 = pltpu.async_copy(ea.at[pl.ds(base, CH)], ob, semg)
        g1.wait()
        g2.wait()
        g3.wait()

        @pl.loop(0, CH)
        def _row(r):
            for j in range(D_FEAT // 16):
                sl = pl.ds(j * 16, 16)
                plsc.addupdate(ob.at[r, sl], gs[r, sl])
                plsc.addupdate(ob.at[r, sl], gr[r, sl])

        w1 = pltpu.async_copy(ob, ne.at[pl.ds(base, CH)], semw)
        w2 = pltpu.async_copy(ob, acc_sh.at[idx_s], sema, add=True)
        w3 = pltpu.async_copy(ones_b, cnt_sh.at[idx_s], semc, add=True)
        w2.wait()
        w3.wait()
        w1.wait()

    plsc.subcore_barrier()
    f1 = pltpu.async_copy(acc_sh.at[pl.ds(rbase, RPW)],
                          accs.at[cid, pl.ds(rbase, RPW)], semz)
    f2 = pltpu.async_copy(cnt_sh.at[pl.ds(rbase, RPW)],
                          cnts.at[cid, pl.ds(rbase, RPW)], semz)
    f1.wait()
    f2.wait()
    z1 = pltpu.async_copy(zacc, acc_sh.at[pl.ds(rbase, RPW)], semz)
    z2 = pltpu.async_copy(zcnt, cnt_sh.at[pl.ds(rbase, RPW)], semz)
    z1.wait()
    z2.wait()
    plsc.subcore_barrier()

    @pl.loop(0, NCHUNK)
    def _chunk_b(ci):
        base = ebase + ci * CH
        i1 = pltpu.async_copy(rcv.at[pl.ds(base, CH)], idx_r, semi)
        i2 = pltpu.async_copy(ne.at[pl.ds(base, CH)], ob, semg)
        i1.wait()
        i2.wait()
        w2 = pltpu.async_copy(ob, acc_sh.at[idx_r], sema, add=True)
        w3 = pltpu.async_copy(ones_b, cnt_sh.at[idx_r], semc, add=True)
        w2.wait()
        w3.wait()

    plsc.subcore_barrier()
    f1 = pltpu.async_copy(acc_sh.at[pl.ds(rbase, RPW)],
                          accr.at[cid, pl.ds(rbase, RPW)], semz)
    f2 = pltpu.async_copy(cnt_sh.at[pl.ds(rbase, RPW)],
                          cntr.at[cid, pl.ds(rbase, RPW)], semz)
    f1.wait()
    f2.wait()


_sc_edges = functools.partial(
    pl.kernel,
    out_type=(
        jax.ShapeDtypeStruct((N_EDGES, D_FEAT), jnp.float32),
        jax.ShapeDtypeStruct((NC, N_NODES, D_FEAT), jnp.float32),
        jax.ShapeDtypeStruct((NC, N_NODES, D_FEAT), jnp.float32),
        jax.ShapeDtypeStruct((NC, N_NODES, 16), jnp.float32),
        jax.ShapeDtypeStruct((NC, N_NODES, 16), jnp.float32),
    ),
    mesh=plsc.VectorSubcoreMesh(core_axis_name="c", subcore_axis_name="s"),
    compiler_params=pltpu.CompilerParams(use_tc_tiling_on_sc=False),
    scratch_types=(
        pltpu.VMEM((CH,), jnp.int32),
        pltpu.VMEM((CH,), jnp.int32),
        pltpu.VMEM((CH, D_FEAT), jnp.float32),
        pltpu.VMEM((CH, D_FEAT), jnp.float32),
        pltpu.VMEM((CH, D_FEAT), jnp.float32),
        pltpu.VMEM((CH, 16), jnp.float32),
        pltpu.VMEM_SHARED((N_NODES, D_FEAT), jnp.float32),
        pltpu.VMEM_SHARED((N_NODES, 16), jnp.float32),
        pltpu.SemaphoreType.DMA,
        pltpu.SemaphoreType.DMA,
        pltpu.SemaphoreType.DMA,
        pltpu.SemaphoreType.DMA,
        pltpu.SemaphoreType.DMA,
        pltpu.SemaphoreType.DMA,
    ),
)(_sc_body)


def kernel(x, edge_attr, senders, receivers, W_e, b_e, W_n, b_n):
    we0 = W_e[:D_EDGE]
    we1 = W_e[D_EDGE:D_EDGE + D_FEAT]
    we2 = W_e[D_EDGE + D_FEAT:]

    xst, xrt = pl.pallas_call(
        _tables_body,
        out_shape=[jax.ShapeDtypeStruct((N_NODES, D_FEAT), jnp.float32)] * 2,
    )(x, we1, we2)

    BE = 8000
    ea = pl.pallas_call(
        _ea_body,
        grid=(N_EDGES // BE,),
        in_specs=[
            pl.BlockSpec((BE, D_EDGE), lambda i: (i, 0)),
            pl.BlockSpec((D_EDGE, D_FEAT), lambda i: (0, 0)),
            pl.BlockSpec((1, D_FEAT), lambda i: (0, 0)),
        ],
        out_specs=pl.BlockSpec((BE, D_FEAT), lambda i: (i, 0)),
        out_shape=jax.ShapeDtypeStruct((N_EDGES, D_FEAT), jnp.float32),
    )(edge_attr, we0, b_e.reshape(1, D_FEAT))

    zacc = jnp.zeros((RPW, D_FEAT), jnp.float32)
    zcnt = jnp.zeros((RPW, 16), jnp.float32)
    ne, accs, accr, cnts, cntr = _sc_edges(
        senders, receivers, ea, xst, xrt, zacc, zcnt)

    BN = 400
    new_nodes = pl.pallas_call(
        _node_body,
        grid=(N_NODES // BN,),
        in_specs=[
            pl.BlockSpec((BN, D_FEAT), lambda i: (i, 0)),
            pl.BlockSpec((NC, BN, D_FEAT), lambda i: (0, i, 0)),
            pl.BlockSpec((NC, BN, D_FEAT), lambda i: (0, i, 0)),
            pl.BlockSpec((NC, BN, 16), lambda i: (0, i, 0)),
            pl.BlockSpec((NC, BN, 16), lambda i: (0, i, 0)),
            pl.BlockSpec((3 * D_FEAT, D_FEAT), lambda i: (0, 0)),
            pl.BlockSpec((1, D_FEAT), lambda i: (0, 0)),
        ],
        out_specs=pl.BlockSpec((BN, D_FEAT), lambda i: (i, 0)),
        out_shape=jax.ShapeDtypeStruct((N_NODES, D_FEAT), jnp.float32),
    )(x, accs, accr, cnts, cntr, W_n, b_n.reshape(1, D_FEAT))

    return new_nodes, ne


# speedup vs baseline: 5.0286x; 1.2822x over previous
import functools

import jax
import jax.numpy as jnp
from jax import lax
from jax.experimental import pallas as pl
from jax.experimental.pallas import tpu as pltpu
from jax.experimental.pallas import tpu_sc as plsc

N_NODES = 10000
N_EDGES = 320000
D_FEAT = 128
D_EDGE = 16
NC = 2
NS = 16
NW = NC * NS
EPW = N_EDGES // NW
CH = 40
CPB = 10
NBLK = EPW // (CH * CPB)
RPW = N_NODES // NS
ZROWS = 25


def _tables_body(x_ref, we1_ref, we2_ref, xs_ref, xr_ref):
    xs_ref[...] = jnp.dot(x_ref[...], we1_ref[...],
                          preferred_element_type=jnp.float32)
    xr_ref[...] = jnp.dot(x_ref[...], we2_ref[...],
                          preferred_element_type=jnp.float32)


def _ea_body(ea_ref, we0_ref, be_ref, o_ref):
    o_ref[...] = jnp.dot(ea_ref[...], we0_ref[...],
                         preferred_element_type=jnp.float32) + be_ref[...]


def _node_body(x_ref, accs_ref, accr_ref, cnts_ref, cntr_ref, wn_ref, bn_ref,
               o_ref):
    ss = accs_ref[0] + accs_ref[1]
    sr = accr_ref[0] + accr_ref[1]
    cs = cnts_ref[0, :, :1] + cnts_ref[1, :, :1]
    cr = cntr_ref[0, :, :1] + cntr_ref[1, :, :1]
    rs = 1.0 / jnp.maximum(cs, 1.0)
    rr = 1.0 / jnp.maximum(cr, 1.0)
    wn = wn_ref[...]
    out = jnp.dot(x_ref[...], wn[:128], preferred_element_type=jnp.float32)
    out += jnp.dot(ss * rs, wn[128:256], preferred_element_type=jnp.float32)
    out += jnp.dot(sr * rr, wn[256:384], preferred_element_type=jnp.float32)
    o_ref[...] = out + bn_ref[...]


def _sc_body(snd2, rcv2, ea, xst, xrt, zacc, zcnt,
             ne, accs, accr, cnts, cntr,
             idxb_s, idxb_r, gs0, gs1, gr0, gr1, ob0, ob1, ones_b,
             acc_sh, cnt_sh, semi, semg, semw, semz, sema, semc):
    cid = lax.axis_index("c")
    sid = lax.axis_index("s")
    wid = sid * NC + cid
    ebase = wid * EPW
    rbase = sid * RPW
    gsl = (gs0, gs1)
    grl = (gr0, gr1)
    obl = (ob0, ob1)

    ones16 = jnp.ones((16,), jnp.float32)

    @pl.loop(0, CH)
    def _zo(r):
        ones_b[r, pl.ds(0, 16)] = ones16

    z1 = pltpu.async_copy(zacc, acc_sh.at[pl.ds(rbase, RPW)], semz)
    z2 = pltpu.async_copy(zcnt, cnt_sh.at[pl.ds(rbase, RPW)], semz)
    z1.wait()
    z2.wait()
    plsc.subcore_barrier()

    def _add_rows(ob, gs, gr):
        @pl.loop(0, CH)
        def _row(r):
            for j in range(D_FEAT // 16):
                sl = pl.ds(j * 16, 16)
                plsc.addupdate(ob.at[r, sl], gs[r, sl])
                plsc.addupdate(ob.at[r, sl], gr[r, sl])

    @pl.loop(0, NBLK)
    def _blk(bi):
        row0 = wid * (NBLK * CPB) + bi * CPB
        i1 = pltpu.async_copy(snd2.at[pl.ds(row0, CPB)], idxb_s, semi)
        i2 = pltpu.async_copy(rcv2.at[pl.ds(row0, CPB)], idxb_r, semi)
        i1.wait()
        i2.wait()

        def _issue_gathers(k, s):
            base = ebase + (bi * CPB + k) * CH
            g1 = pltpu.async_copy(xst.at[idxb_s.at[k]], gsl[s], semg)
            g2 = pltpu.async_copy(xrt.at[idxb_r.at[k]], grl[s], semg)
            g3 = pltpu.async_copy(ea.at[pl.ds(base, CH)], obl[s], semg)
            return (g1, g2, g3)

        gd = [None, None]
        wd = [None, None]
        gd[0] = _issue_gathers(0, 0)
        for k in range(CPB):
            s = k % 2
            if k < CPB - 1:
                if k >= 1:
                    for w in wd[1 - s]:
                        w.wait()
                gd[1 - s] = _issue_gathers(k + 1, 1 - s)
            for g in gd[s]:
                g.wait()
            _add_rows(obl[s], gsl[s], grl[s])
            base = ebase + (bi * CPB + k) * CH
            w1 = pltpu.async_copy(obl[s], ne.at[pl.ds(base, CH)], semw)
            w2 = pltpu.async_copy(obl[s], acc_sh.at[idxb_s.at[k]], sema,
                                  add=True)
            w3 = pltpu.async_copy(ones_b, cnt_sh.at[idxb_s.at[k]], semc,
                                  add=True)
            wd[s] = (w1, w2, w3)
        for s in (0, 1):
            for w in wd[s]:
                w.wait()

    plsc.subcore_barrier()
    f1 = pltpu.async_copy(acc_sh.at[pl.ds(rbase, RPW)],
                          accs.at[cid, pl.ds(rbase, RPW)], semz)
    f2 = pltpu.async_copy(cnt_sh.at[pl.ds(rbase, RPW)],
                          cnts.at[cid, pl.ds(rbase, RPW)], semz)
    f1.wait()
    f2.wait()
    z1 = pltpu.async_copy(zacc, acc_sh.at[pl.ds(rbase, RPW)], semz)
    z2 = pltpu.async_copy(zcnt, cnt_sh.at[pl.ds(rbase, RPW)], semz)
    z1.wait()
    z2.wait()
    plsc.subcore_barrier()

    @pl.loop(0, NBLK)
    def _blk_b(bi):
        row0 = wid * (NBLK * CPB) + bi * CPB
        i1 = pltpu.async_copy(rcv2.at[pl.ds(row0, CPB)], idxb_r, semi)
        i1.wait()

        def _issue_read(k, s):
            base = ebase + (bi * CPB + k) * CH
            return pltpu.async_copy(ne.at[pl.ds(base, CH)], obl[s], semg)

        rd = [None, None]
        wd = [None, None]
        rd[0] = _issue_read(0, 0)
        for k in range(CPB):
            s = k % 2
            if k < CPB - 1:
                if k >= 1:
                    for w in wd[1 - s]:
                        w.wait()
                rd[1 - s] = _issue_read(k + 1, 1 - s)
            rd[s].wait()
            w2 = pltpu.async_copy(obl[s], acc_sh.at[idxb_r.at[k]], sema,
                                  add=True)
            w3 = pltpu.async_copy(ones_b, cnt_sh.at[idxb_r.at[k]], semc,
                                  add=True)
            wd[s] = (w2, w3)
        for s in (0, 1):
            for w in wd[s]:
                w.wait()

    plsc.subcore_barrier()
    f1 = pltpu.async_copy(acc_sh.at[pl.ds(rbase, RPW)],
                          accr.at[cid, pl.ds(rbase, RPW)], semz)
    f2 = pltpu.async_copy(cnt_sh.at[pl.ds(rbase, RPW)],
                          cntr.at[cid, pl.ds(rbase, RPW)], semz)
    f1.wait()
    f2.wait()


_sc_edges = functools.partial(
    pl.kernel,
    out_type=(
        jax.ShapeDtypeStruct((N_EDGES, D_FEAT), jnp.float32),
        jax.ShapeDtypeStruct((NC, N_NODES, D_FEAT), jnp.float32),
        jax.ShapeDtypeStruct((NC, N_NODES, D_FEAT), jnp.float32),
        jax.ShapeDtypeStruct((NC, N_NODES, 16), jnp.float32),
        jax.ShapeDtypeStruct((NC, N_NODES, 16), jnp.float32),
    ),
    mesh=plsc.VectorSubcoreMesh(core_axis_name="c", subcore_axis_name="s"),
    compiler_params=pltpu.CompilerParams(use_tc_tiling_on_sc=False),
    scratch_types=(
        pltpu.VMEM((CPB, CH), jnp.int32),
        pltpu.VMEM((CPB, CH), jnp.int32),
        pltpu.VMEM((CH, D_FEAT), jnp.float32),
        pltpu.VMEM((CH, D_FEAT), jnp.float32),
        pltpu.VMEM((CH, D_FEAT), jnp.float32),
        pltpu.VMEM((CH, D_FEAT), jnp.float32),
        pltpu.VMEM((CH, D_FEAT), jnp.float32),
        pltpu.VMEM((CH, D_FEAT), jnp.float32),
        pltpu.VMEM((CH, 16), jnp.float32),
        pltpu.VMEM_SHARED((N_NODES, D_FEAT), jnp.float32),
        pltpu.VMEM_SHARED((N_NODES, 16), jnp.float32),
        pltpu.SemaphoreType.DMA,
        pltpu.SemaphoreType.DMA,
        pltpu.SemaphoreType.DMA,
        pltpu.SemaphoreType.DMA,
        pltpu.SemaphoreType.DMA,
        pltpu.SemaphoreType.DMA,
    ),
)(_sc_body)


def kernel(x, edge_attr, senders, receivers, W_e, b_e, W_n, b_n):
    we0 = W_e[:D_EDGE]
    we1 = W_e[D_EDGE:D_EDGE + D_FEAT]
    we2 = W_e[D_EDGE + D_FEAT:]

    xst, xrt = pl.pallas_call(
        _tables_body,
        out_shape=[jax.ShapeDtypeStruct((N_NODES, D_FEAT), jnp.float32)] * 2,
    )(x, we1, we2)

    BE = 8000
    ea = pl.pallas_call(
        _ea_body,
        grid=(N_EDGES // BE,),
        in_specs=[
            pl.BlockSpec((BE, D_EDGE), lambda i: (i, 0)),
            pl.BlockSpec((D_EDGE, D_FEAT), lambda i: (0, 0)),
            pl.BlockSpec((1, D_FEAT), lambda i: (0, 0)),
        ],
        out_specs=pl.BlockSpec((BE, D_FEAT), lambda i: (i, 0)),
        out_shape=jax.ShapeDtypeStruct((N_EDGES, D_FEAT), jnp.float32),
    )(edge_attr, we0, b_e.reshape(1, D_FEAT))

    zacc = jnp.zeros((RPW, D_FEAT), jnp.float32)
    zcnt = jnp.zeros((RPW, 16), jnp.float32)
    ne, accs, accr, cnts, cntr = _sc_edges(
        senders.reshape(N_EDGES // CH, CH), receivers.reshape(N_EDGES // CH, CH),
        ea, xst, xrt, zacc, zcnt)

    BN = 400
    new_nodes = pl.pallas_call(
        _node_body,
        grid=(N_NODES // BN,),
        in_specs=[
            pl.BlockSpec((BN, D_FEAT), lambda i: (i, 0)),
            pl.BlockSpec((NC, BN, D_FEAT), lambda i: (0, i, 0)),
            pl.BlockSpec((NC, BN, D_FEAT), lambda i: (0, i, 0)),
            pl.BlockSpec((NC, BN, 16), lambda i: (0, i, 0)),
            pl.BlockSpec((NC, BN, 16), lambda i: (0, i, 0)),
            pl.BlockSpec((3 * D_FEAT, D_FEAT), lambda i: (0, 0)),
            pl.BlockSpec((1, D_FEAT), lambda i: (0, 0)),
        ],
        out_specs=pl.BlockSpec((BN, D_FEAT), lambda i: (i, 0)),
        out_shape=jax.ShapeDtypeStruct((N_NODES, D_FEAT), jnp.float32),
    )(x, accs, accr, cnts, cntr, W_n, b_n.reshape(1, D_FEAT))

    return new_nodes, ne


# speedup vs baseline: 5.2586x; 1.0457x over previous
import functools

import jax
import jax.numpy as jnp
from jax import lax
from jax.experimental import pallas as pl
from jax.experimental.pallas import tpu as pltpu
from jax.experimental.pallas import tpu_sc as plsc

N_NODES = 10000
N_EDGES = 320000
D_FEAT = 128
D_EDGE = 16
NC = 2
NS = 16
NW = NC * NS
EPW = N_EDGES // NW
CH = 40
CPB = 25
NBLK = EPW // (CH * CPB)
RPW = N_NODES // NS
ZROWS = 25


def _tables_body(x_ref, we1_ref, we2_ref, xs_ref, xr_ref):
    xs_ref[...] = jnp.dot(x_ref[...], we1_ref[...],
                          preferred_element_type=jnp.float32)
    xr_ref[...] = jnp.dot(x_ref[...], we2_ref[...],
                          preferred_element_type=jnp.float32)


def _ea_body(ea_ref, we0_ref, be_ref, o_ref):
    o_ref[...] = jnp.dot(ea_ref[...], we0_ref[...],
                         preferred_element_type=jnp.float32) + be_ref[...]


def _node_body(x_ref, accs_ref, accr_ref, cnts_ref, cntr_ref, wn_ref, bn_ref,
               o_ref):
    ss = accs_ref[0] + accs_ref[1]
    sr = accr_ref[0] + accr_ref[1]
    cs = cnts_ref[0, :, :1] + cnts_ref[1, :, :1]
    cr = cntr_ref[0, :, :1] + cntr_ref[1, :, :1]
    rs = 1.0 / jnp.maximum(cs, 1.0)
    rr = 1.0 / jnp.maximum(cr, 1.0)
    wn = wn_ref[...]
    out = jnp.dot(x_ref[...], wn[:128], preferred_element_type=jnp.float32)
    out += jnp.dot(ss * rs, wn[128:256], preferred_element_type=jnp.float32)
    out += jnp.dot(sr * rr, wn[256:384], preferred_element_type=jnp.float32)
    o_ref[...] = out + bn_ref[...]


def _sc_body(snd2, rcv2, ea, xst, xrt, zacc, zcnt,
             ne, accs, accr, cnts, cntr,
             idxb_s, idxb_r, gs0, gs1, gr0, gr1, ob0, ob1, ones_b,
             acc_sh, cnt_sh, semi, semg, semw, semz, sema, semc):
    cid = lax.axis_index("c")
    sid = lax.axis_index("s")
    wid = sid * NC + cid
    ebase = wid * EPW
    rbase = sid * RPW
    gsl = (gs0, gs1)
    grl = (gr0, gr1)
    obl = (ob0, ob1)

    ones16 = jnp.ones((16,), jnp.float32)

    @pl.loop(0, CH)
    def _zo(r):
        ones_b[r, pl.ds(0, 16)] = ones16

    z1 = pltpu.async_copy(zacc, acc_sh.at[pl.ds(rbase, RPW)], semz)
    z2 = pltpu.async_copy(zcnt, cnt_sh.at[pl.ds(rbase, RPW)], semz)
    z1.wait()
    z2.wait()
    plsc.subcore_barrier()

    def _add_rows(ob, gs, gr):
        @pl.loop(0, CH)
        def _row(r):
            for j in range(D_FEAT // 16):
                sl = pl.ds(j * 16, 16)
                plsc.addupdate(ob.at[r, sl], gs[r, sl])
                plsc.addupdate(ob.at[r, sl], gr[r, sl])

    @pl.loop(0, NBLK)
    def _blk(bi):
        row0 = wid * (NBLK * CPB) + bi * CPB
        i1 = pltpu.async_copy(snd2.at[pl.ds(row0, CPB)], idxb_s, semi)
        i2 = pltpu.async_copy(rcv2.at[pl.ds(row0, CPB)], idxb_r, semi)
        i1.wait()
        i2.wait()

        def _issue_gathers(k, s):
            base = ebase + (bi * CPB + k) * CH
            g1 = pltpu.async_copy(xst.at[idxb_s.at[k]], gsl[s], semg)
            g2 = pltpu.async_copy(xrt.at[idxb_r.at[k]], grl[s], semg)
            g3 = pltpu.async_copy(ea.at[pl.ds(base, CH)], obl[s], semg)
            return (g1, g2, g3)

        gd = [None, None]
        wd = [None, None]
        gd[0] = _issue_gathers(0, 0)
        for k in range(CPB):
            s = k % 2
            if k < CPB - 1:
                if k >= 1:
                    for w in wd[1 - s]:
                        w.wait()
                gd[1 - s] = _issue_gathers(k + 1, 1 - s)
            for g in gd[s]:
                g.wait()
            _add_rows(obl[s], gsl[s], grl[s])
            base = ebase + (bi * CPB + k) * CH
            w1 = pltpu.async_copy(obl[s], ne.at[pl.ds(base, CH)], semw)
            w2 = pltpu.async_copy(obl[s], acc_sh.at[idxb_s.at[k]], sema,
                                  add=True)
            w3 = pltpu.async_copy(ones_b, cnt_sh.at[idxb_s.at[k]], semc,
                                  add=True)
            wd[s] = (w1, w2, w3)
        for s in (0, 1):
            for w in wd[s]:
                w.wait()

    plsc.subcore_barrier()
    f1 = pltpu.async_copy(acc_sh.at[pl.ds(rbase, RPW)],
                          accs.at[cid, pl.ds(rbase, RPW)], semz)
    f2 = pltpu.async_copy(cnt_sh.at[pl.ds(rbase, RPW)],
                          cnts.at[cid, pl.ds(rbase, RPW)], semz)
    f1.wait()
    f2.wait()
    z1 = pltpu.async_copy(zacc, acc_sh.at[pl.ds(rbase, RPW)], semz)
    z2 = pltpu.async_copy(zcnt, cnt_sh.at[pl.ds(rbase, RPW)], semz)
    z1.wait()
    z2.wait()
    plsc.subcore_barrier()

    @pl.loop(0, NBLK)
    def _blk_b(bi):
        row0 = wid * (NBLK * CPB) + bi * CPB
        i1 = pltpu.async_copy(rcv2.at[pl.ds(row0, CPB)], idxb_r, semi)
        i1.wait()

        def _issue_read(k, s):
            base = ebase + (bi * CPB + k) * CH
            return pltpu.async_copy(ne.at[pl.ds(base, CH)], obl[s], semg)

        rd = [None, None]
        wd = [None, None]
        rd[0] = _issue_read(0, 0)
        for k in range(CPB):
            s = k % 2
            if k < CPB - 1:
                if k >= 1:
                    for w in wd[1 - s]:
                        w.wait()
                rd[1 - s] = _issue_read(k + 1, 1 - s)
            rd[s].wait()
            w2 = pltpu.async_copy(obl[s], acc_sh.at[idxb_r.at[k]], sema,
                                  add=True)
            w3 = pltpu.async_copy(ones_b, cnt_sh.at[idxb_r.at[k]], semc,
                                  add=True)
            wd[s] = (w2, w3)
        for s in (0, 1):
            for w in wd[s]:
                w.wait()

    plsc.subcore_barrier()
    f1 = pltpu.async_copy(acc_sh.at[pl.ds(rbase, RPW)],
                          accr.at[cid, pl.ds(rbase, RPW)], semz)
    f2 = pltpu.async_copy(cnt_sh.at[pl.ds(rbase, RPW)],
                          cntr.at[cid, pl.ds(rbase, RPW)], semz)
    f1.wait()
    f2.wait()


_sc_edges = functools.partial(
    pl.kernel,
    out_type=(
        jax.ShapeDtypeStruct((N_EDGES, D_FEAT), jnp.float32),
        jax.ShapeDtypeStruct((NC, N_NODES, D_FEAT), jnp.float32),
        jax.ShapeDtypeStruct((NC, N_NODES, D_FEAT), jnp.float32),
        jax.ShapeDtypeStruct((NC, N_NODES, 16), jnp.float32),
        jax.ShapeDtypeStruct((NC, N_NODES, 16), jnp.float32),
    ),
    mesh=plsc.VectorSubcoreMesh(core_axis_name="c", subcore_axis_name="s"),
    compiler_params=pltpu.CompilerParams(use_tc_tiling_on_sc=False),
    scratch_types=(
        pltpu.VMEM((CPB, CH), jnp.int32),
        pltpu.VMEM((CPB, CH), jnp.int32),
        pltpu.VMEM((CH, D_FEAT), jnp.float32),
        pltpu.VMEM((CH, D_FEAT), jnp.float32),
        pltpu.VMEM((CH, D_FEAT), jnp.float32),
        pltpu.VMEM((CH, D_FEAT), jnp.float32),
        pltpu.VMEM((CH, D_FEAT), jnp.float32),
        pltpu.VMEM((CH, D_FEAT), jnp.float32),
        pltpu.VMEM((CH, 16), jnp.float32),
        pltpu.VMEM_SHARED((N_NODES, D_FEAT), jnp.float32),
        pltpu.VMEM_SHARED((N_NODES, 16), jnp.float32),
        pltpu.SemaphoreType.DMA,
        pltpu.SemaphoreType.DMA,
        pltpu.SemaphoreType.DMA,
        pltpu.SemaphoreType.DMA,
        pltpu.SemaphoreType.DMA,
        pltpu.SemaphoreType.DMA,
    ),
)(_sc_body)


def kernel(x, edge_attr, senders, receivers, W_e, b_e, W_n, b_n):
    we0 = W_e[:D_EDGE]
    we1 = W_e[D_EDGE:D_EDGE + D_FEAT]
    we2 = W_e[D_EDGE + D_FEAT:]

    xst, xrt = pl.pallas_call(
        _tables_body,
        out_shape=[jax.ShapeDtypeStruct((N_NODES, D_FEAT), jnp.float32)] * 2,
    )(x, we1, we2)

    BE = 16000
    ea = pl.pallas_call(
        _ea_body,
        grid=(N_EDGES // BE,),
        in_specs=[
            pl.BlockSpec((BE, D_EDGE), lambda i: (i, 0)),
            pl.BlockSpec((D_EDGE, D_FEAT), lambda i: (0, 0)),
            pl.BlockSpec((1, D_FEAT), lambda i: (0, 0)),
        ],
        out_specs=pl.BlockSpec((BE, D_FEAT), lambda i: (i, 0)),
        out_shape=jax.ShapeDtypeStruct((N_EDGES, D_FEAT), jnp.float32),
    )(edge_attr, we0, b_e.reshape(1, D_FEAT))

    zacc = jnp.zeros((RPW, D_FEAT), jnp.float32)
    zcnt = jnp.zeros((RPW, 16), jnp.float32)
    ne, accs, accr, cnts, cntr = _sc_edges(
        senders.reshape(N_EDGES // CH, CH), receivers.reshape(N_EDGES // CH, CH),
        ea, xst, xrt, zacc, zcnt)

    BN = 400
    new_nodes = pl.pallas_call(
        _node_body,
        grid=(N_NODES // BN,),
        in_specs=[
            pl.BlockSpec((BN, D_FEAT), lambda i: (i, 0)),
            pl.BlockSpec((NC, BN, D_FEAT), lambda i: (0, i, 0)),
            pl.BlockSpec((NC, BN, D_FEAT), lambda i: (0, i, 0)),
            pl.BlockSpec((NC, BN, 16), lambda i: (0, i, 0)),
            pl.BlockSpec((NC, BN, 16), lambda i: (0, i, 0)),
            pl.BlockSpec((3 * D_FEAT, D_FEAT), lambda i: (0, 0)),
            pl.BlockSpec((1, D_FEAT), lambda i: (0, 0)),
        ],
        out_specs=pl.BlockSpec((BN, D_FEAT), lambda i: (i, 0)),
        out_shape=jax.ShapeDtypeStruct((N_NODES, D_FEAT), jnp.float32),
    )(x, accs, accr, cnts, cntr, W_n, b_n.reshape(1, D_FEAT))

    return new_nodes, ne


# speedup vs baseline: 6.2351x; 1.1857x over previous
import functools

import jax
import jax.numpy as jnp
from jax import lax
from jax.experimental import pallas as pl
from jax.experimental.pallas import tpu as pltpu
from jax.experimental.pallas import tpu_sc as plsc

N_NODES = 10000
N_EDGES = 320000
D_FEAT = 128
D_EDGE = 16
NC = 2
NS = 16
NW = NC * NS
EPW = N_EDGES // NW
CH = 40
CPB = 25
NBLK = EPW // (CH * CPB)
RPW = N_NODES // NS
ZROWS = 25


def _tables_body(x_ref, we1_ref, we2_ref, xs_ref, xr_ref):
    xs_ref[...] = jnp.dot(x_ref[...], we1_ref[...],
                          preferred_element_type=jnp.float32)
    xr_ref[...] = jnp.dot(x_ref[...], we2_ref[...],
                          preferred_element_type=jnp.float32)


def _ea_body(et_ref, we0_ref, be_ref, o_ref):
    o_ref[...] = lax.dot_general(
        et_ref[...], we0_ref[...], (((0,), (0,)), ((), ())),
        preferred_element_type=jnp.float32) + be_ref[...]


def _node_body(x_ref, accs_ref, accr_ref, cnts_ref, cntr_ref, wn_ref, bn_ref,
               o_ref):
    ss = accs_ref[0] + accs_ref[1]
    sr = accr_ref[0] + accr_ref[1]
    cs = cnts_ref[0, :, :1] + cnts_ref[1, :, :1]
    cr = cntr_ref[0, :, :1] + cntr_ref[1, :, :1]
    rs = 1.0 / jnp.maximum(cs, 1.0)
    rr = 1.0 / jnp.maximum(cr, 1.0)
    wn = wn_ref[...]
    out = jnp.dot(x_ref[...], wn[:128], preferred_element_type=jnp.float32)
    out += jnp.dot(ss * rs, wn[128:256], preferred_element_type=jnp.float32)
    out += jnp.dot(sr * rr, wn[256:384], preferred_element_type=jnp.float32)
    o_ref[...] = out + bn_ref[...]


def _sc_body(snd2, rcv2, ea, xst, xrt, zacc, zcnt,
             ne, accs, accr, cnts, cntr,
             idxb_s, idxb_r, gs0, gs1, gr0, gr1, ob0, ob1, ones_b,
             acc_sh, cnt_sh, semi, semg, semw, semz, sema, semc):
    cid = lax.axis_index("c")
    sid = lax.axis_index("s")
    wid = sid * NC + cid
    ebase = wid * EPW
    rbase = sid * RPW
    gsl = (gs0, gs1)
    grl = (gr0, gr1)
    obl = (ob0, ob1)

    ones16 = jnp.ones((16,), jnp.float32)

    @pl.loop(0, CH)
    def _zo(r):
        ones_b[r, pl.ds(0, 16)] = ones16

    z1 = pltpu.async_copy(zacc, acc_sh.at[pl.ds(rbase, RPW)], semz)
    z2 = pltpu.async_copy(zcnt, cnt_sh.at[pl.ds(rbase, RPW)], semz)
    z1.wait()
    z2.wait()
    plsc.subcore_barrier()

    def _add_rows(ob, gs, gr):
        @pl.loop(0, CH)
        def _row(r):
            for j in range(D_FEAT // 16):
                sl = pl.ds(j * 16, 16)
                plsc.addupdate(ob.at[r, sl], gs[r, sl])
                plsc.addupdate(ob.at[r, sl], gr[r, sl])

    @pl.loop(0, NBLK)
    def _blk(bi):
        row0 = wid * (NBLK * CPB) + bi * CPB
        i1 = pltpu.async_copy(snd2.at[pl.ds(row0, CPB)], idxb_s, semi)
        i2 = pltpu.async_copy(rcv2.at[pl.ds(row0, CPB)], idxb_r, semi)
        i1.wait()
        i2.wait()

        def _issue_gathers(k, s):
            base = ebase + (bi * CPB + k) * CH
            g1 = pltpu.async_copy(xst.at[idxb_s.at[k]], gsl[s], semg)
            g2 = pltpu.async_copy(xrt.at[idxb_r.at[k]], grl[s], semg)
            g3 = pltpu.async_copy(ea.at[pl.ds(base, CH)], obl[s], semg)
            return (g1, g2, g3)

        gd = [None, None]
        wd = [None, None]
        gd[0] = _issue_gathers(0, 0)
        for k in range(CPB):
            s = k % 2
            if k < CPB - 1:
                if k >= 1:
                    for w in wd[1 - s]:
                        w.wait()
                gd[1 - s] = _issue_gathers(k + 1, 1 - s)
            for g in gd[s]:
                g.wait()
            _add_rows(obl[s], gsl[s], grl[s])
            base = ebase + (bi * CPB + k) * CH
            w1 = pltpu.async_copy(obl[s], ne.at[pl.ds(base, CH)], semw)
            w2 = pltpu.async_copy(obl[s], acc_sh.at[idxb_s.at[k]], sema,
                                  add=True)
            w3 = pltpu.async_copy(ones_b, cnt_sh.at[idxb_s.at[k]], semc,
                                  add=True)
            wd[s] = (w1, w2, w3)
        for s in (0, 1):
            for w in wd[s]:
                w.wait()

    plsc.subcore_barrier()
    f1 = pltpu.async_copy(acc_sh.at[pl.ds(rbase, RPW)],
                          accs.at[cid, pl.ds(rbase, RPW)], semz)
    f2 = pltpu.async_copy(cnt_sh.at[pl.ds(rbase, RPW)],
                          cnts.at[cid, pl.ds(rbase, RPW)], semz)
    f1.wait()
    f2.wait()
    z1 = pltpu.async_copy(zacc, acc_sh.at[pl.ds(rbase, RPW)], semz)
    z2 = pltpu.async_copy(zcnt, cnt_sh.at[pl.ds(rbase, RPW)], semz)
    z1.wait()
    z2.wait()
    plsc.subcore_barrier()

    @pl.loop(0, NBLK)
    def _blk_b(bi):
        row0 = wid * (NBLK * CPB) + bi * CPB
        i1 = pltpu.async_copy(rcv2.at[pl.ds(row0, CPB)], idxb_r, semi)
        i1.wait()

        def _issue_read(k, s):
            base = ebase + (bi * CPB + k) * CH
            return pltpu.async_copy(ne.at[pl.ds(base, CH)], obl[s], semg)

        rd = [None, None]
        wd = [None, None]
        rd[0] = _issue_read(0, 0)
        for k in range(CPB):
            s = k % 2
            if k < CPB - 1:
                if k >= 1:
                    for w in wd[1 - s]:
                        w.wait()
                rd[1 - s] = _issue_read(k + 1, 1 - s)
            rd[s].wait()
            w2 = pltpu.async_copy(obl[s], acc_sh.at[idxb_r.at[k]], sema,
                                  add=True)
            w3 = pltpu.async_copy(ones_b, cnt_sh.at[idxb_r.at[k]], semc,
                                  add=True)
            wd[s] = (w2, w3)
        for s in (0, 1):
            for w in wd[s]:
                w.wait()

    plsc.subcore_barrier()
    f1 = pltpu.async_copy(acc_sh.at[pl.ds(rbase, RPW)],
                          accr.at[cid, pl.ds(rbase, RPW)], semz)
    f2 = pltpu.async_copy(cnt_sh.at[pl.ds(rbase, RPW)],
                          cntr.at[cid, pl.ds(rbase, RPW)], semz)
    f1.wait()
    f2.wait()


_sc_edges = functools.partial(
    pl.kernel,
    out_type=(
        jax.ShapeDtypeStruct((N_EDGES, D_FEAT), jnp.float32),
        jax.ShapeDtypeStruct((NC, N_NODES, D_FEAT), jnp.float32),
        jax.ShapeDtypeStruct((NC, N_NODES, D_FEAT), jnp.float32),
        jax.ShapeDtypeStruct((NC, N_NODES, 16), jnp.float32),
        jax.ShapeDtypeStruct((NC, N_NODES, 16), jnp.float32),
    ),
    mesh=plsc.VectorSubcoreMesh(core_axis_name="c", subcore_axis_name="s"),
    compiler_params=pltpu.CompilerParams(use_tc_tiling_on_sc=False),
    scratch_types=(
        pltpu.VMEM((CPB, CH), jnp.int32),
        pltpu.VMEM((CPB, CH), jnp.int32),
        pltpu.VMEM((CH, D_FEAT), jnp.float32),
        pltpu.VMEM((CH, D_FEAT), jnp.float32),
        pltpu.VMEM((CH, D_FEAT), jnp.float32),
        pltpu.VMEM((CH, D_FEAT), jnp.float32),
        pltpu.VMEM((CH, D_FEAT), jnp.float32),
        pltpu.VMEM((CH, D_FEAT), jnp.float32),
        pltpu.VMEM((CH, 16), jnp.float32),
        pltpu.VMEM_SHARED((N_NODES, D_FEAT), jnp.float32),
        pltpu.VMEM_SHARED((N_NODES, 16), jnp.float32),
        pltpu.SemaphoreType.DMA,
        pltpu.SemaphoreType.DMA,
        pltpu.SemaphoreType.DMA,
        pltpu.SemaphoreType.DMA,
        pltpu.SemaphoreType.DMA,
        pltpu.SemaphoreType.DMA,
    ),
)(_sc_body)


def kernel(x, edge_attr, senders, receivers, W_e, b_e, W_n, b_n):
    we0 = W_e[:D_EDGE]
    we1 = W_e[D_EDGE:D_EDGE + D_FEAT]
    we2 = W_e[D_EDGE + D_FEAT:]

    xst, xrt = pl.pallas_call(
        _tables_body,
        out_shape=[jax.ShapeDtypeStruct((N_NODES, D_FEAT), jnp.float32)] * 2,
    )(x, we1, we2)

    BE = 16000
    ea = pl.pallas_call(
        _ea_body,
        grid=(N_EDGES // BE,),
        in_specs=[
            pl.BlockSpec((D_EDGE, BE), lambda i: (0, i)),
            pl.BlockSpec((D_EDGE, D_FEAT), lambda i: (0, 0)),
            pl.BlockSpec((1, D_FEAT), lambda i: (0, 0)),
        ],
        out_specs=pl.BlockSpec((BE, D_FEAT), lambda i: (i, 0)),
        out_shape=jax.ShapeDtypeStruct((N_EDGES, D_FEAT), jnp.float32),
    )(edge_attr.T, we0, b_e.reshape(1, D_FEAT))

    zacc = jnp.zeros((RPW, D_FEAT), jnp.float32)
    zcnt = jnp.zeros((RPW, 16), jnp.float32)
    ne, accs, accr, cnts, cntr = _sc_edges(
        senders.reshape(N_EDGES // CH, CH), receivers.reshape(N_EDGES // CH, CH),
        ea, xst, xrt, zacc, zcnt)

    BN = 400
    new_nodes = pl.pallas_call(
        _node_body,
        grid=(N_NODES // BN,),
        in_specs=[
            pl.BlockSpec((BN, D_FEAT), lambda i: (i, 0)),
            pl.BlockSpec((NC, BN, D_FEAT), lambda i: (0, i, 0)),
            pl.BlockSpec((NC, BN, D_FEAT), lambda i: (0, i, 0)),
            pl.BlockSpec((NC, BN, 16), lambda i: (0, i, 0)),
            pl.BlockSpec((NC, BN, 16), lambda i: (0, i, 0)),
            pl.BlockSpec((3 * D_FEAT, D_FEAT), lambda i: (0, 0)),
            pl.BlockSpec((1, D_FEAT), lambda i: (0, 0)),
        ],
        out_specs=pl.BlockSpec((BN, D_FEAT), lambda i: (i, 0)),
        out_shape=jax.ShapeDtypeStruct((N_NODES, D_FEAT), jnp.float32),
    )(x, accs, accr, cnts, cntr, W_n, b_n.reshape(1, D_FEAT))

    return new_nodes, ne
